# baseline (speedup 1.0000x reference)
import functools
import math

import jax
import jax.numpy as jnp
from jax import lax
from jax.experimental import pallas as pl
from jax.experimental.pallas import tpu as pltpu
from jax.experimental.pallas import tpu_sc as plsc

_INV_SQRT2 = 1.0 / math.sqrt(2.0)

_N_ATOMS = 10000
_N_EDGES = 320000
_D = 128
_DR = 16

_N_CORES = 2
_N_SUB = 16

_CH = 256
_SG = 128
_N_CHUNKS = _N_EDGES // _CH
_HALF = _N_ATOMS // _N_CORES
_ACC_ROWS = _HALF + 8
_TRASH = _HALF
_N_GROUPS = _HALF // 8
_GRP_BASE = _N_GROUPS // _N_SUB
_GRP_REM = _N_GROUPS % _N_SUB
_TROWS = _GRP_BASE * 8
_MAXCH = 80
_IDXBUF = _MAXCH * _CH
_IDX_PAD = 96 * _CH

_PREC = lax.Precision.DEFAULT


def _silu(v):
    return v * (1.0 / (1.0 + jnp.exp(-v)))


def _res_block(v, w1, w2):
    v2 = _silu(lax.dot(v, w1, precision=_PREC))
    v2 = _silu(lax.dot(v2, w2, precision=_PREC))
    return (v + v2) * _INV_SQRT2


def _edge_body(m_ref, b_ref, wrbf_ref, wrbff_ref, wf_ref, x_ref, xf_ref):
    mm = m_ref[...]
    bb = b_ref[...]
    emb_e = lax.dot(bb, wrbf_ref[...], precision=_PREC)
    x_ref[...] = mm * emb_e
    t = mm
    for i in range(2):
        t = _res_block(t, wf_ref[i, 0], wf_ref[i, 1])
    emb_f = lax.dot(bb, wrbff_ref[...], precision=_PREC)
    xf_ref[...] = t * emb_f


def _atom_body(p_ref, h_ref, wep_ref, we2_ref, o_ref):
    v = p_ref[...]
    for i in range(2):
        v = _res_block(v, wep_ref[i, 0], wep_ref[i, 1])
    v = (v + h_ref[...]) * _INV_SQRT2
    for i in range(2):
        v = _res_block(v, we2_ref[i, 0], we2_ref[i, 1])
    o_ref[...] = v


def _edge_pass(m, basis_rad, w_rbf, w_rbf_f, wf):
    be = 10000
    grid = (_N_EDGES // be,)
    return pl.pallas_call(
        _edge_body,
        grid=grid,
        in_specs=[
            pl.BlockSpec((be, _D), lambda i: (i, 0)),
            pl.BlockSpec((be, _DR), lambda i: (i, 0)),
            pl.BlockSpec((_DR, _D), lambda i: (0, 0)),
            pl.BlockSpec((_DR, _D), lambda i: (0, 0)),
            pl.BlockSpec((2, 2, _D, _D), lambda i: (0, 0, 0, 0)),
        ],
        out_specs=[
            pl.BlockSpec((be, _D), lambda i: (i, 0)),
            pl.BlockSpec((be, _D), lambda i: (i, 0)),
        ],
        out_shape=[
            jax.ShapeDtypeStruct((_N_EDGES, _D), jnp.float32),
            jax.ShapeDtypeStruct((_N_EDGES, _D), jnp.float32),
        ],
        compiler_params=pltpu.CompilerParams(
            dimension_semantics=("parallel",)),
    )(m, basis_rad, w_rbf, w_rbf_f, wf)


def _atom_pass(seg, h, wep, we2):
    ba = 1000
    grid = (_N_ATOMS // ba,)
    return pl.pallas_call(
        _atom_body,
        grid=grid,
        in_specs=[
            pl.BlockSpec((ba, _D), lambda i: (i, 0)),
            pl.BlockSpec((ba, _D), lambda i: (i, 0)),
            pl.BlockSpec((2, 2, _D, _D), lambda i: (0, 0, 0, 0)),
            pl.BlockSpec((2, 2, _D, _D), lambda i: (0, 0, 0, 0)),
        ],
        out_specs=pl.BlockSpec((ba, _D), lambda i: (i, 0)),
        out_shape=jax.ShapeDtypeStruct((_N_ATOMS, _D), jnp.float32),
        compiler_params=pltpu.CompilerParams(
            dimension_semantics=("parallel",)),
    )(seg, h, wep, we2)


def _seg_sum_body(x_hbm, idxp_hbm, par_hbm, out_hbm,
                  idx_all, pbuf, iv00, iv01, iv10, iv11,
                  xbuf0, xbuf1, sem_l0, sem_l1, sem_s0, sem_s1, acc):
    cid = lax.axis_index("c")
    sid = lax.axis_index("s")
    lo = cid * _HALF
    hi = lo + _HALF

    pltpu.sync_copy(par_hbm, pbuf)
    pv = pbuf[pl.ds(0, 16)]
    start = jnp.where(cid == 0, pv[0], pv[2])
    count = jnp.where(cid == 0, pv[1], pv[3])
    per = (count + _N_SUB - 1) // _N_SUB
    n2 = (per + 1) // 2

    tbase = start + sid * per

    def _xrows(k):
        c = jnp.minimum(tbase + k, _N_CHUNKS - 1)
        return x_hbm.at[pl.ds(c * _CH, _CH)]

    pltpu.sync_copy(idxp_hbm.at[pl.ds(tbase * _CH, _IDXBUF)], idx_all)
    pltpu.async_copy(_xrows(0), xbuf0, sem_l0)

    z = jnp.zeros((16,), jnp.float32)

    def zrow(i, carry):
        for j in range(_D // 16):
            xbuf1[i, pl.ds(j * 16, 16)] = z
        return carry

    lax.fori_loop(0, _CH, zrow, 0)
    row0 = (sid * _GRP_BASE + jnp.minimum(sid, _GRP_REM)) * 8
    pltpu.sync_copy(xbuf1, acc.at[pl.ds(row0, _CH)])
    pltpu.sync_copy(xbuf1.at[pl.ds(0, _TROWS - _CH)],
                    acc.at[pl.ds(row0 + _CH, _TROWS - _CH)])

    @pl.when(sid < _GRP_REM)
    def _zero_extra():
        pltpu.sync_copy(xbuf1.at[pl.ds(0, 8)],
                        acc.at[pl.ds(row0 + _TROWS, 8)])

    plsc.subcore_barrier()

    def _mask(k, hi_eff, iv_refs):
        loff = k * _CH
        for g in range(_CH // _SG):
            for j in range(_SG // 16):
                v = idx_all[pl.ds(loff + g * _SG + j * 16, 16)]
                ok = jnp.logical_and(v >= lo, v < hi_eff)
                iv_refs[g][pl.ds(j * 16, 16)] = jnp.where(ok, v - lo, _TRASH)

    def pair_body(i, carry):
        for b in range(2):
            k = 2 * i + b
            xb = xbuf0 if b == 0 else xbuf1
            xo = xbuf1 if b == 0 else xbuf0
            sem_l = sem_l0 if b == 0 else sem_l1
            sem_lo_ = sem_l1 if b == 0 else sem_l0
            sem_s = sem_s0 if b == 0 else sem_s1
            sem_so = sem_s1 if b == 0 else sem_s0
            ivs = (iv00, iv01) if b == 0 else (iv10, iv11)
            pltpu.make_async_copy(_xrows(k), xb, sem_l).wait()
            if b == 0:
                @pl.when(i > 0)
                def _drain():
                    pltpu.make_async_copy(_xrows(k), xo, sem_so).wait()
            else:
                pltpu.make_async_copy(_xrows(k), xo, sem_so).wait()
            pltpu.async_copy(_xrows(k + 1), xo, sem_lo_)
            kvalid = jnp.logical_and(k < per, sid * per + k < count)
            hi_eff = jnp.where(kvalid, hi, lo)
            _mask(k, hi_eff, ivs)
            for g in range(_CH // _SG):
                pltpu.async_copy(xb.at[pl.ds(g * _SG, _SG)],
                                 acc.at[ivs[g]], sem_s, add=True)
        return carry

    lax.fori_loop(0, n2, pair_body, 0)
    pltpu.make_async_copy(_xrows(0), xbuf0, sem_l0).wait()

    @pl.when(n2 > 0)
    def _drain_tail():
        pltpu.make_async_copy(_xrows(0), xbuf1, sem_s1).wait()

    plsc.subcore_barrier()

    pltpu.sync_copy(acc.at[pl.ds(row0, _CH)],
                    out_hbm.at[pl.ds(lo + row0, _CH)])
    pltpu.sync_copy(acc.at[pl.ds(row0 + _CH, _TROWS - _CH)],
                    out_hbm.at[pl.ds(lo + row0 + _CH, _TROWS - _CH)])

    @pl.when(sid < _GRP_REM)
    def _dump_extra():
        pltpu.sync_copy(acc.at[pl.ds(row0 + _TROWS, 8)],
                        out_hbm.at[pl.ds(lo + row0 + _TROWS, 8)])


@functools.lru_cache(maxsize=1)
def _make_seg_sum():
    mesh = plsc.VectorSubcoreMesh(core_axis_name="c", subcore_axis_name="s")
    return functools.partial(
        pl.kernel,
        mesh=mesh,
        out_type=jax.ShapeDtypeStruct((_N_ATOMS, _D), jnp.float32),
        scratch_types=[
            pltpu.VMEM((_IDXBUF,), jnp.int32),
            pltpu.VMEM((16,), jnp.int32),
            pltpu.VMEM((_SG,), jnp.int32),
            pltpu.VMEM((_SG,), jnp.int32),
            pltpu.VMEM((_SG,), jnp.int32),
            pltpu.VMEM((_SG,), jnp.int32),
            pltpu.VMEM((_CH, _D), jnp.float32),
            pltpu.VMEM((_CH, _D), jnp.float32),
            pltpu.SemaphoreType.DMA,
            pltpu.SemaphoreType.DMA,
            pltpu.SemaphoreType.DMA,
            pltpu.SemaphoreType.DMA,
            pltpu.VMEM_SHARED((_ACC_ROWS, _D), jnp.float32),
        ],
    )(_seg_sum_body)


def _chunk_params(idx_atom):
    ends = idx_atom.reshape(_N_CHUNKS, _CH)
    count0 = jnp.searchsorted(ends[:, 0], _HALF).astype(jnp.int32)
    start1 = jnp.searchsorted(ends[:, _CH - 1], _HALF).astype(jnp.int32)
    par = jnp.zeros((16,), jnp.int32)
    par = par.at[1].set(count0)
    par = par.at[2].set(start1)
    par = par.at[3].set(_N_CHUNKS - start1)
    idx_padded = jnp.concatenate(
        [idx_atom, jnp.zeros((_IDX_PAD,), jnp.int32)])
    return idx_padded, par


def kernel(h, m, basis_rad, idx_atom, W_rbf, Wep, We2, Wf, W_rbf_F):
    x_edge, x_f = _edge_pass(m, basis_rad, W_rbf, W_rbf_F, Wf)
    idx_padded, par = _chunk_params(idx_atom)
    seg = _make_seg_sum()(x_edge, idx_padded, par)
    x_e = _atom_pass(seg, h, Wep, We2)
    return (x_e, x_f)


# speedup vs baseline: 1.0094x; 1.0094x over previous
import functools
import math

import jax
import jax.numpy as jnp
from jax import lax
from jax.experimental import pallas as pl
from jax.experimental.pallas import tpu as pltpu
from jax.experimental.pallas import tpu_sc as plsc

_INV_SQRT2 = 1.0 / math.sqrt(2.0)

_N_ATOMS = 10000
_N_EDGES = 320000
_D = 128
_DR = 16

_N_CORES = 2
_N_SUB = 16

_CH = 256
_SG = 128
_N_CHUNKS = _N_EDGES // _CH
_HALF = _N_ATOMS // _N_CORES
_ACC_ROWS = _HALF + 8
_TRASH = _HALF
_N_GROUPS = _HALF // 8
_GRP_BASE = _N_GROUPS // _N_SUB
_GRP_REM = _N_GROUPS % _N_SUB
_TROWS = _GRP_BASE * 8
_MAXCH = 80
_IDXBUF = _MAXCH * _CH
_IDX_PAD = 96 * _CH

_PREC = lax.Precision.DEFAULT


def _silu(v):
    return v * (1.0 / (1.0 + jnp.exp(-v)))


def _res_block(v, w1, w2):
    v2 = _silu(lax.dot(v, w1, precision=_PREC))
    v2 = _silu(lax.dot(v2, w2, precision=_PREC))
    return (v + v2) * _INV_SQRT2


def _edge_body(m_ref, b_ref, wrbf_ref, wrbff_ref, wf_ref, x_ref, xf_ref):
    mm = m_ref[...]
    bb = b_ref[...]
    emb_e = lax.dot(bb, wrbf_ref[...], precision=_PREC)
    x_ref[...] = mm * emb_e
    t = mm
    for i in range(2):
        t = _res_block(t, wf_ref[i, 0], wf_ref[i, 1])
    emb_f = lax.dot(bb, wrbff_ref[...], precision=_PREC)
    xf_ref[...] = t * emb_f


def _atom_body(p_ref, h_ref, wep_ref, we2_ref, o_ref):
    v = p_ref[...]
    for i in range(2):
        v = _res_block(v, wep_ref[i, 0], wep_ref[i, 1])
    v = (v + h_ref[...]) * _INV_SQRT2
    for i in range(2):
        v = _res_block(v, we2_ref[i, 0], we2_ref[i, 1])
    o_ref[...] = v


def _edge_pass(m, basis_rad, w_rbf, w_rbf_f, wf):
    be = 10000
    grid = (_N_EDGES // be,)
    return pl.pallas_call(
        _edge_body,
        grid=grid,
        in_specs=[
            pl.BlockSpec((be, _D), lambda i: (i, 0)),
            pl.BlockSpec((be, _DR), lambda i: (i, 0)),
            pl.BlockSpec((_DR, _D), lambda i: (0, 0)),
            pl.BlockSpec((_DR, _D), lambda i: (0, 0)),
            pl.BlockSpec((2, 2, _D, _D), lambda i: (0, 0, 0, 0)),
        ],
        out_specs=[
            pl.BlockSpec((be, _D), lambda i: (i, 0)),
            pl.BlockSpec((be, _D), lambda i: (i, 0)),
        ],
        out_shape=[
            jax.ShapeDtypeStruct((_N_EDGES, _D), jnp.float32),
            jax.ShapeDtypeStruct((_N_EDGES, _D), jnp.float32),
        ],
        compiler_params=pltpu.CompilerParams(
            dimension_semantics=("parallel",)),
    )(m, basis_rad, w_rbf, w_rbf_f, wf)


def _atom_pass(seg, h, wep, we2):
    ba = 2000
    grid = (_N_ATOMS // ba,)
    return pl.pallas_call(
        _atom_body,
        grid=grid,
        in_specs=[
            pl.BlockSpec((ba, _D), lambda i: (i, 0)),
            pl.BlockSpec((ba, _D), lambda i: (i, 0)),
            pl.BlockSpec((2, 2, _D, _D), lambda i: (0, 0, 0, 0)),
            pl.BlockSpec((2, 2, _D, _D), lambda i: (0, 0, 0, 0)),
        ],
        out_specs=pl.BlockSpec((ba, _D), lambda i: (i, 0)),
        out_shape=jax.ShapeDtypeStruct((_N_ATOMS, _D), jnp.float32),
        compiler_params=pltpu.CompilerParams(
            dimension_semantics=("parallel",)),
    )(seg, h, wep, we2)


def _seg_sum_body(x_hbm, idxp_hbm, par_hbm, out_hbm,
                  idx_all, pbuf, iv00, iv01, iv10, iv11,
                  xbuf0, xbuf1, sem_l0, sem_l1, sem_s0, sem_s1, acc):
    cid = lax.axis_index("c")
    sid = lax.axis_index("s")
    lo = cid * _HALF
    hi = lo + _HALF

    pltpu.sync_copy(par_hbm, pbuf)
    pv = pbuf[pl.ds(0, 16)]
    start = jnp.where(cid == 0, pv[0], pv[2])
    count = jnp.where(cid == 0, pv[1], pv[3])
    per = (count + _N_SUB - 1) // _N_SUB
    n2 = (per + 1) // 2

    tbase = start + sid * per

    def _xrows(k):
        c = jnp.minimum(tbase + k, _N_CHUNKS - 1)
        return x_hbm.at[pl.ds(c * _CH, _CH)]

    pltpu.sync_copy(idxp_hbm.at[pl.ds(tbase * _CH, _IDXBUF)], idx_all)
    pltpu.async_copy(_xrows(0), xbuf0, sem_l0)

    z = jnp.zeros((16,), jnp.float32)

    def zrow(i, carry):
        for j in range(_D // 16):
            xbuf1[i, pl.ds(j * 16, 16)] = z
        return carry

    lax.fori_loop(0, _CH, zrow, 0)
    row0 = (sid * _GRP_BASE + jnp.minimum(sid, _GRP_REM)) * 8
    pltpu.sync_copy(xbuf1, acc.at[pl.ds(row0, _CH)])
    pltpu.sync_copy(xbuf1.at[pl.ds(0, _TROWS - _CH)],
                    acc.at[pl.ds(row0 + _CH, _TROWS - _CH)])

    @pl.when(sid < _GRP_REM)
    def _zero_extra():
        pltpu.sync_copy(xbuf1.at[pl.ds(0, 8)],
                        acc.at[pl.ds(row0 + _TROWS, 8)])

    plsc.subcore_barrier()

    def _mask(k, hi_eff, iv_refs):
        loff = k * _CH
        for g in range(_CH // _SG):
            for j in range(_SG // 16):
                v = idx_all[pl.ds(loff + g * _SG + j * 16, 16)]
                ok = jnp.logical_and(v >= lo, v < hi_eff)
                iv_refs[g][pl.ds(j * 16, 16)] = jnp.where(ok, v - lo, _TRASH)

    def pair_body(i, carry):
        for b in range(2):
            k = 2 * i + b
            xb = xbuf0 if b == 0 else xbuf1
            xo = xbuf1 if b == 0 else xbuf0
            sem_l = sem_l0 if b == 0 else sem_l1
            sem_lo_ = sem_l1 if b == 0 else sem_l0
            sem_s = sem_s0 if b == 0 else sem_s1
            sem_so = sem_s1 if b == 0 else sem_s0
            ivs = (iv00, iv01) if b == 0 else (iv10, iv11)
            pltpu.make_async_copy(_xrows(k), xb, sem_l).wait()
            if b == 0:
                @pl.when(i > 0)
                def _drain():
                    pltpu.make_async_copy(_xrows(k), xo, sem_so).wait()
            else:
                pltpu.make_async_copy(_xrows(k), xo, sem_so).wait()
            pltpu.async_copy(_xrows(k + 1), xo, sem_lo_)
            kvalid = jnp.logical_and(k < per, sid * per + k < count)
            hi_eff = jnp.where(kvalid, hi, lo)
            _mask(k, hi_eff, ivs)
            for g in range(_CH // _SG):
                pltpu.async_copy(xb.at[pl.ds(g * _SG, _SG)],
                                 acc.at[ivs[g]], sem_s, add=True)
        return carry

    lax.fori_loop(0, n2, pair_body, 0)
    pltpu.make_async_copy(_xrows(0), xbuf0, sem_l0).wait()

    @pl.when(n2 > 0)
    def _drain_tail():
        pltpu.make_async_copy(_xrows(0), xbuf1, sem_s1).wait()

    plsc.subcore_barrier()

    pltpu.sync_copy(acc.at[pl.ds(row0, _CH)],
                    out_hbm.at[pl.ds(lo + row0, _CH)])
    pltpu.sync_copy(acc.at[pl.ds(row0 + _CH, _TROWS - _CH)],
                    out_hbm.at[pl.ds(lo + row0 + _CH, _TROWS - _CH)])

    @pl.when(sid < _GRP_REM)
    def _dump_extra():
        pltpu.sync_copy(acc.at[pl.ds(row0 + _TROWS, 8)],
                        out_hbm.at[pl.ds(lo + row0 + _TROWS, 8)])


@functools.lru_cache(maxsize=1)
def _make_seg_sum():
    mesh = plsc.VectorSubcoreMesh(core_axis_name="c", subcore_axis_name="s")
    return functools.partial(
        pl.kernel,
        mesh=mesh,
        out_type=jax.ShapeDtypeStruct((_N_ATOMS, _D), jnp.float32),
        scratch_types=[
            pltpu.VMEM((_IDXBUF,), jnp.int32),
            pltpu.VMEM((16,), jnp.int32),
            pltpu.VMEM((_SG,), jnp.int32),
            pltpu.VMEM((_SG,), jnp.int32),
            pltpu.VMEM((_SG,), jnp.int32),
            pltpu.VMEM((_SG,), jnp.int32),
            pltpu.VMEM((_CH, _D), jnp.float32),
            pltpu.VMEM((_CH, _D), jnp.float32),
            pltpu.SemaphoreType.DMA,
            pltpu.SemaphoreType.DMA,
            pltpu.SemaphoreType.DMA,
            pltpu.SemaphoreType.DMA,
            pltpu.VMEM_SHARED((_ACC_ROWS, _D), jnp.float32),
        ],
    )(_seg_sum_body)


def _chunk_params(idx_atom):
    ends = idx_atom.reshape(_N_CHUNKS, _CH)
    count0 = jnp.searchsorted(ends[:, 0], _HALF).astype(jnp.int32)
    start1 = jnp.searchsorted(ends[:, _CH - 1], _HALF).astype(jnp.int32)
    par = jnp.zeros((16,), jnp.int32)
    par = par.at[1].set(count0)
    par = par.at[2].set(start1)
    par = par.at[3].set(_N_CHUNKS - start1)
    idx_padded = jnp.concatenate(
        [idx_atom, jnp.zeros((_IDX_PAD,), jnp.int32)])
    return idx_padded, par


def kernel(h, m, basis_rad, idx_atom, W_rbf, Wep, We2, Wf, W_rbf_F):
    x_edge, x_f = _edge_pass(m, basis_rad, W_rbf, W_rbf_F, Wf)
    idx_padded, par = _chunk_params(idx_atom)
    seg = _make_seg_sum()(x_edge, idx_padded, par)
    x_e = _atom_pass(seg, h, Wep, We2)
    return (x_e, x_f)
